# baseline (speedup 1.0000x reference)
import jax
import jax.numpy as jnp
from jax.experimental import pallas as pl
from jax.experimental.pallas import tpu as pltpu


def _phase1(adj_ref, dist_ref, cos_ref, feat_ref, w1_ref, b1_ref, wa_ref,
            ba_ref, w2a_ref, w2b_ref, y_ref, deg_ref):
    feat = feat_ref[...].astype(jnp.bfloat16)
    adj = adj_ref[...]
    deg = jnp.sum(adj, axis=1, keepdims=True) + 1e-6
    s1 = jnp.dot(adj.astype(jnp.bfloat16), feat,
                 preferred_element_type=jnp.float32)
    w = jnp.exp(-dist_ref[...]) * (1.0 + cos_ref[...])
    wsum = jnp.sum(w, axis=1, keepdims=True) + 1e-6
    s2 = jnp.dot(w.astype(jnp.bfloat16), feat,
                 preferred_element_type=jnp.float32)
    xc = jnp.maximum(
        jnp.dot(s1 / deg, w1_ref[...], preferred_element_type=jnp.float32)
        + b1_ref[...], 0.0)
    xa = jnp.maximum(
        jnp.dot(s2 / wsum, wa_ref[...], preferred_element_type=jnp.float32)
        + ba_ref[...], 0.0)
    y_ref[...] = (
        jnp.dot(xc, w2a_ref[...], preferred_element_type=jnp.float32)
        + jnp.dot(xa, w2b_ref[...], preferred_element_type=jnp.float32))
    deg_ref[...] = deg


def _phase2(adj_ref, y_ref, deg_ref, b2_ref, out_ref):
    out_ref[...] = (
        jnp.dot(adj_ref[...].astype(jnp.bfloat16),
                y_ref[...].astype(jnp.bfloat16),
                preferred_element_type=jnp.float32)
        / deg_ref[...] + b2_ref[...])


def kernel(features, adj, dist, adj_relative_cos, W1, b1, Wa, ba, W2, b2):
    n, d_in = features.shape
    d_hid = W1.shape[1]
    d_out = W2.shape[1]
    w2a = W2[:d_hid]
    w2b = W2[d_hid:]
    b1r = b1.reshape(1, d_hid)
    bar = ba.reshape(1, d_in)
    b2r = b2.reshape(1, d_out)

    tm = min(128, n)
    y, deg = pl.pallas_call(
        _phase1,
        grid=(pl.cdiv(n, tm),),
        in_specs=[
            pl.BlockSpec((tm, n), lambda i: (i, 0)),
            pl.BlockSpec((tm, n), lambda i: (i, 0)),
            pl.BlockSpec((tm, n), lambda i: (i, 0)),
            pl.BlockSpec((n, d_in), lambda i: (0, 0)),
            pl.BlockSpec((d_in, d_hid), lambda i: (0, 0)),
            pl.BlockSpec((1, d_hid), lambda i: (0, 0)),
            pl.BlockSpec((d_in, d_in), lambda i: (0, 0)),
            pl.BlockSpec((1, d_in), lambda i: (0, 0)),
            pl.BlockSpec((d_hid, d_out), lambda i: (0, 0)),
            pl.BlockSpec((d_in, d_out), lambda i: (0, 0)),
        ],
        out_specs=[
            pl.BlockSpec((tm, d_out), lambda i: (i, 0)),
            pl.BlockSpec((tm, 1), lambda i: (i, 0)),
        ],
        out_shape=[
            jax.ShapeDtypeStruct((n, d_out), jnp.float32),
            jax.ShapeDtypeStruct((n, 1), jnp.float32),
        ],
        compiler_params=pltpu.CompilerParams(vmem_limit_bytes=62 * 1024 * 1024),
    )(adj, dist, adj_relative_cos, features, W1, b1r, Wa, bar, w2a, w2b)

    tm2 = min(512, n)
    out = pl.pallas_call(
        _phase2,
        grid=(pl.cdiv(n, tm2),),
        in_specs=[
            pl.BlockSpec((tm2, n), lambda i: (i, 0)),
            pl.BlockSpec((n, d_out), lambda i: (0, 0)),
            pl.BlockSpec((tm2, 1), lambda i: (i, 0)),
            pl.BlockSpec((1, d_out), lambda i: (0, 0)),
        ],
        out_specs=pl.BlockSpec((tm2, d_out), lambda i: (i, 0)),
        out_shape=jax.ShapeDtypeStruct((n, d_out), jnp.float32),
        compiler_params=pltpu.CompilerParams(vmem_limit_bytes=62 * 1024 * 1024),
    )(adj, y, deg, b2r)
    return out


# speedup vs baseline: 1.0783x; 1.0783x over previous
import jax
import jax.numpy as jnp
from jax.experimental import pallas as pl
from jax.experimental.pallas import tpu as pltpu


def _phase1(adj_ref, dist_ref, cos_ref, feat_ref, w1_ref, b1_ref, wa_ref,
            ba_ref, w2a_ref, w2b_ref, y_ref, deg_ref, adjq_ref):
    feat = feat_ref[...].astype(jnp.bfloat16)
    adj = adj_ref[...]
    deg = jnp.sum(adj, axis=1, keepdims=True) + 1e-6
    s1 = jnp.dot(adj.astype(jnp.bfloat16), feat,
                 preferred_element_type=jnp.float32)
    adjq_ref[...] = (adj * 127.0 + 0.5).astype(jnp.int8)
    w = jnp.exp(-dist_ref[...]) * (1.0 + cos_ref[...])
    wsum = jnp.sum(w, axis=1, keepdims=True) + 1e-6
    s2 = jnp.dot(w.astype(jnp.bfloat16), feat,
                 preferred_element_type=jnp.float32)
    xc = jnp.maximum(
        jnp.dot(s1 / deg, w1_ref[...], preferred_element_type=jnp.float32)
        + b1_ref[...], 0.0)
    xa = jnp.maximum(
        jnp.dot(s2 / wsum, wa_ref[...], preferred_element_type=jnp.float32)
        + ba_ref[...], 0.0)
    y = (jnp.dot(xc, w2a_ref[...], preferred_element_type=jnp.float32)
         + jnp.dot(xa, w2b_ref[...], preferred_element_type=jnp.float32))
    y_ref[...] = (y * (1.0 / 127.0)).astype(jnp.bfloat16)
    deg_ref[...] = deg


def _phase2(adjq_ref, y_ref, deg_ref, b2_ref, out_ref):
    out_ref[...] = (
        jnp.dot(adjq_ref[...].astype(jnp.bfloat16), y_ref[...],
                preferred_element_type=jnp.float32)
        / deg_ref[...] + b2_ref[...])


def kernel(features, adj, dist, adj_relative_cos, W1, b1, Wa, ba, W2, b2):
    n, d_in = features.shape
    d_hid = W1.shape[1]
    d_out = W2.shape[1]
    w2a = W2[:d_hid]
    w2b = W2[d_hid:]
    b1r = b1.reshape(1, d_hid)
    bar = ba.reshape(1, d_in)
    b2r = b2.reshape(1, d_out)

    tm = min(128, n)
    y, deg, adj_q = pl.pallas_call(
        _phase1,
        grid=(pl.cdiv(n, tm),),
        in_specs=[
            pl.BlockSpec((tm, n), lambda i: (i, 0)),
            pl.BlockSpec((tm, n), lambda i: (i, 0)),
            pl.BlockSpec((tm, n), lambda i: (i, 0)),
            pl.BlockSpec((n, d_in), lambda i: (0, 0)),
            pl.BlockSpec((d_in, d_hid), lambda i: (0, 0)),
            pl.BlockSpec((1, d_hid), lambda i: (0, 0)),
            pl.BlockSpec((d_in, d_in), lambda i: (0, 0)),
            pl.BlockSpec((1, d_in), lambda i: (0, 0)),
            pl.BlockSpec((d_hid, d_out), lambda i: (0, 0)),
            pl.BlockSpec((d_in, d_out), lambda i: (0, 0)),
        ],
        out_specs=[
            pl.BlockSpec((tm, d_out), lambda i: (i, 0)),
            pl.BlockSpec((tm, 1), lambda i: (i, 0)),
            pl.BlockSpec((tm, n), lambda i: (i, 0)),
        ],
        out_shape=[
            jax.ShapeDtypeStruct((n, d_out), jnp.bfloat16),
            jax.ShapeDtypeStruct((n, 1), jnp.float32),
            jax.ShapeDtypeStruct((n, n), jnp.int8),
        ],
        compiler_params=pltpu.CompilerParams(vmem_limit_bytes=62 * 1024 * 1024),
    )(adj, dist, adj_relative_cos, features, W1, b1r, Wa, bar, w2a, w2b)

    tm2 = min(512, n)
    out = pl.pallas_call(
        _phase2,
        grid=(pl.cdiv(n, tm2),),
        in_specs=[
            pl.BlockSpec((tm2, n), lambda i: (i, 0)),
            pl.BlockSpec((n, d_out), lambda i: (0, 0)),
            pl.BlockSpec((tm2, 1), lambda i: (i, 0)),
            pl.BlockSpec((1, d_out), lambda i: (0, 0)),
        ],
        out_specs=pl.BlockSpec((tm2, d_out), lambda i: (i, 0)),
        out_shape=jax.ShapeDtypeStruct((n, d_out), jnp.float32),
        compiler_params=pltpu.CompilerParams(vmem_limit_bytes=62 * 1024 * 1024),
    )(adj_q, y, deg, b2r)
    return out


# speedup vs baseline: 1.0850x; 1.0063x over previous
import jax
import jax.numpy as jnp
from jax.experimental import pallas as pl
from jax.experimental.pallas import tpu as pltpu


def _phase1(adj_ref, dist_ref, cos_ref, feat_ref, w1_ref, b1_ref, wa_ref,
            ba_ref, w2a_ref, w2b_ref, y_ref, deg_ref, adjq_ref):
    feat = feat_ref[...].astype(jnp.bfloat16)
    adj = adj_ref[...]
    deg = jnp.sum(adj, axis=1, keepdims=True) + 1e-6
    s1 = jnp.dot(adj.astype(jnp.bfloat16), feat,
                 preferred_element_type=jnp.float32)
    adjq_ref[...] = (adj * 127.0 + 0.5).astype(jnp.int8)
    w = jnp.exp(-dist_ref[...]) * (1.0 + cos_ref[...])
    wsum = jnp.sum(w, axis=1, keepdims=True) + 1e-6
    s2 = jnp.dot(w.astype(jnp.bfloat16), feat,
                 preferred_element_type=jnp.float32)
    xc = jnp.maximum(
        jnp.dot(s1 / deg, w1_ref[...], preferred_element_type=jnp.float32)
        + b1_ref[...], 0.0)
    xa = jnp.maximum(
        jnp.dot(s2 / wsum, wa_ref[...], preferred_element_type=jnp.float32)
        + ba_ref[...], 0.0)
    y = (jnp.dot(xc, w2a_ref[...], preferred_element_type=jnp.float32)
         + jnp.dot(xa, w2b_ref[...], preferred_element_type=jnp.float32))
    y_ref[...] = (y * (1.0 / 127.0)).astype(jnp.bfloat16)
    deg_ref[...] = deg


def _phase2(adjq_ref, y_ref, deg_ref, b2_ref, out_ref):
    out_ref[...] = (
        jnp.dot(adjq_ref[...].astype(jnp.bfloat16), y_ref[...],
                preferred_element_type=jnp.float32)
        / deg_ref[...] + b2_ref[...])


def kernel(features, adj, dist, adj_relative_cos, W1, b1, Wa, ba, W2, b2):
    n, d_in = features.shape
    d_hid = W1.shape[1]
    d_out = W2.shape[1]
    w2a = W2[:d_hid]
    w2b = W2[d_hid:]
    b1r = b1.reshape(1, d_hid)
    bar = ba.reshape(1, d_in)
    b2r = b2.reshape(1, d_out)

    tm = min(128, n)
    y, deg, adj_q = pl.pallas_call(
        _phase1,
        grid=(pl.cdiv(n, tm),),
        in_specs=[
            pl.BlockSpec((tm, n), lambda i: (i, 0)),
            pl.BlockSpec((tm, n), lambda i: (i, 0)),
            pl.BlockSpec((tm, n), lambda i: (i, 0)),
            pl.BlockSpec((n, d_in), lambda i: (0, 0)),
            pl.BlockSpec((d_in, d_hid), lambda i: (0, 0)),
            pl.BlockSpec((1, d_hid), lambda i: (0, 0)),
            pl.BlockSpec((d_in, d_in), lambda i: (0, 0)),
            pl.BlockSpec((1, d_in), lambda i: (0, 0)),
            pl.BlockSpec((d_hid, d_out), lambda i: (0, 0)),
            pl.BlockSpec((d_in, d_out), lambda i: (0, 0)),
        ],
        out_specs=[
            pl.BlockSpec((tm, d_out), lambda i: (i, 0)),
            pl.BlockSpec((tm, 1), lambda i: (i, 0)),
            pl.BlockSpec((tm, n), lambda i: (i, 0)),
        ],
        out_shape=[
            jax.ShapeDtypeStruct((n, d_out), jnp.bfloat16),
            jax.ShapeDtypeStruct((n, 1), jnp.float32),
            jax.ShapeDtypeStruct((n, n), jnp.int8),
        ],
        compiler_params=pltpu.CompilerParams(vmem_limit_bytes=62 * 1024 * 1024),
    )(adj, dist, adj_relative_cos, features, W1, b1r, Wa, bar, w2a, w2b)

    tm2 = min(1000, n)
    out = pl.pallas_call(
        _phase2,
        grid=(pl.cdiv(n, tm2),),
        in_specs=[
            pl.BlockSpec((tm2, n), lambda i: (i, 0)),
            pl.BlockSpec((n, d_out), lambda i: (0, 0)),
            pl.BlockSpec((tm2, 1), lambda i: (i, 0)),
            pl.BlockSpec((1, d_out), lambda i: (0, 0)),
        ],
        out_specs=pl.BlockSpec((tm2, d_out), lambda i: (i, 0)),
        out_shape=jax.ShapeDtypeStruct((n, d_out), jnp.float32),
        compiler_params=pltpu.CompilerParams(vmem_limit_bytes=62 * 1024 * 1024),
    )(adj_q, y, deg, b2r)
    return out


# speedup vs baseline: 1.0876x; 1.0024x over previous
import jax
import jax.numpy as jnp
from jax.experimental import pallas as pl
from jax.experimental.pallas import tpu as pltpu


def _phase1(adj_ref, dist_ref, cos_ref, feat_ref, w1_ref, b1_ref, wa_ref,
            ba_ref, w2a_ref, w2b_ref, y_ref, deg_ref, adjq_ref):
    feat = feat_ref[...].astype(jnp.bfloat16)
    adj = adj_ref[...]
    deg = jnp.sum(adj, axis=1, keepdims=True) + 1e-6
    s1 = jnp.dot(adj.astype(jnp.bfloat16), feat,
                 preferred_element_type=jnp.float32)
    adjq_ref[...] = (adj * 127.0 + 0.5).astype(jnp.int8)
    w = jnp.exp(-dist_ref[...]) * (1.0 + cos_ref[...])
    wsum = jnp.sum(w, axis=1, keepdims=True) + 1e-6
    s2 = jnp.dot(w.astype(jnp.bfloat16), feat,
                 preferred_element_type=jnp.float32)
    xc = jnp.maximum(
        jnp.dot(s1 / deg, w1_ref[...], preferred_element_type=jnp.float32)
        + b1_ref[...], 0.0)
    xa = jnp.maximum(
        jnp.dot(s2 / wsum, wa_ref[...], preferred_element_type=jnp.float32)
        + ba_ref[...], 0.0)
    y = (jnp.dot(xc, w2a_ref[...], preferred_element_type=jnp.float32)
         + jnp.dot(xa, w2b_ref[...], preferred_element_type=jnp.float32))
    y_ref[...] = (y * (1.0 / 127.0)).astype(jnp.bfloat16)
    deg_ref[...] = deg


def _phase2(adjq_ref, y_ref, deg_ref, b2_ref, out_ref):
    out_ref[...] = (
        jnp.dot(adjq_ref[...].astype(jnp.bfloat16), y_ref[...],
                preferred_element_type=jnp.float32)
        / deg_ref[...] + b2_ref[...])


def kernel(features, adj, dist, adj_relative_cos, W1, b1, Wa, ba, W2, b2):
    n, d_in = features.shape
    d_hid = W1.shape[1]
    d_out = W2.shape[1]
    w2a = W2[:d_hid]
    w2b = W2[d_hid:]
    b1r = b1.reshape(1, d_hid)
    bar = ba.reshape(1, d_in)
    b2r = b2.reshape(1, d_out)

    tm = min(200, n)
    y, deg, adj_q = pl.pallas_call(
        _phase1,
        grid=(pl.cdiv(n, tm),),
        in_specs=[
            pl.BlockSpec((tm, n), lambda i: (i, 0)),
            pl.BlockSpec((tm, n), lambda i: (i, 0)),
            pl.BlockSpec((tm, n), lambda i: (i, 0)),
            pl.BlockSpec((n, d_in), lambda i: (0, 0)),
            pl.BlockSpec((d_in, d_hid), lambda i: (0, 0)),
            pl.BlockSpec((1, d_hid), lambda i: (0, 0)),
            pl.BlockSpec((d_in, d_in), lambda i: (0, 0)),
            pl.BlockSpec((1, d_in), lambda i: (0, 0)),
            pl.BlockSpec((d_hid, d_out), lambda i: (0, 0)),
            pl.BlockSpec((d_in, d_out), lambda i: (0, 0)),
        ],
        out_specs=[
            pl.BlockSpec((tm, d_out), lambda i: (i, 0)),
            pl.BlockSpec((tm, 1), lambda i: (i, 0)),
            pl.BlockSpec((tm, n), lambda i: (i, 0)),
        ],
        out_shape=[
            jax.ShapeDtypeStruct((n, d_out), jnp.bfloat16),
            jax.ShapeDtypeStruct((n, 1), jnp.float32),
            jax.ShapeDtypeStruct((n, n), jnp.int8),
        ],
        compiler_params=pltpu.CompilerParams(vmem_limit_bytes=62 * 1024 * 1024),
    )(adj, dist, adj_relative_cos, features, W1, b1r, Wa, bar, w2a, w2b)

    tm2 = min(512, n)
    out = pl.pallas_call(
        _phase2,
        grid=(pl.cdiv(n, tm2),),
        in_specs=[
            pl.BlockSpec((tm2, n), lambda i: (i, 0)),
            pl.BlockSpec((n, d_out), lambda i: (0, 0)),
            pl.BlockSpec((tm2, 1), lambda i: (i, 0)),
            pl.BlockSpec((1, d_out), lambda i: (0, 0)),
        ],
        out_specs=pl.BlockSpec((tm2, d_out), lambda i: (i, 0)),
        out_shape=jax.ShapeDtypeStruct((n, d_out), jnp.float32),
        compiler_params=pltpu.CompilerParams(vmem_limit_bytes=62 * 1024 * 1024),
    )(adj_q, y, deg, b2r)
    return out
